# Initial kernel scaffold; baseline (speedup 1.0000x reference)
import jax
import jax.numpy as jnp
from jax.experimental import pallas as pl


def kernel(x, edge_index, W1, b1, W2, b2, W3, b3, g1, be1, g2, be2, val_min, val_max):
    raise NotImplementedError("write your pallas kernel here")


# speedup vs baseline: 165.0301x; 165.0301x over previous
import functools

import jax
import jax.numpy as jnp
from jax import lax
from jax.experimental import pallas as pl
from jax.experimental.pallas import tpu as pltpu
from jax.experimental.pallas import tpu_sc as plsc

NN = 50000
EE = 800000
BB = 2
NP = 51200
EP = 802816
NS = NP // 16
CPT = EP // 128 // 16
PCH = 64
SCH = 4
BLK = 1024

_mesh = plsc.VectorSubcoreMesh(core_axis_name="c", subcore_axis_name="s")


def _fill(ref, rows, width, val):
    v = jnp.full((16,), val, jnp.float32)

    def body(r, _):
        for h in range(width // 16):
            ref[r, pl.ds(h * 16, 16)] = v
        return 0

    lax.fori_loop(0, rows, body, 0)


@functools.partial(
    pl.kernel,
    out_type=jax.ShapeDtypeStruct((2 * NP, 16), jnp.float32),
    mesh=_mesh,
    scratch_types=[
        pltpu.VMEM_SHARED((NP, 16), jnp.float32),
        pltpu.VMEM((4, 128), jnp.int32),
        pltpu.VMEM((128, 16), jnp.float32),
        pltpu.VMEM((PCH, 16), jnp.float32),
        pltpu.SemaphoreType.DMA,
    ],
    compiler_params=pltpu.CompilerParams(use_tc_tiling_on_sc=False),
)
def _deg_kernel(col2d, degp, dacc, cbuf, ones, zb, sem):
    c = lax.axis_index("c")
    s = lax.axis_index("s")
    _fill(ones, 128, 16, 1.0)
    _fill(zb, PCH, 16, 0.0)

    def zero(i, _):
        pltpu.sync_copy(zb, dacc.at[pl.ds(s * NS + i * PCH, PCH)])
        return 0

    lax.fori_loop(0, NS // PCH, zero, 0)
    plsc.subcore_barrier()

    base = c * (EP // 256) + s * (EP // 256 // 16)

    def stage(t, _):
        pltpu.sync_copy(col2d.at[pl.ds(base + t * 4, 4)], cbuf)
        hs = [
            pltpu.async_copy(ones, dacc.at[cbuf.at[j]], sem, add=True)
            for j in range(4)
        ]
        for h in hs:
            h.wait()
        return 0

    lax.fori_loop(0, 49, stage, 0)
    plsc.subcore_barrier()

    def out(i, _):
        g = s * NS + i * PCH
        pltpu.sync_copy(dacc.at[pl.ds(g, PCH)], zb)
        pltpu.sync_copy(zb, degp.at[pl.ds(c * NP + g, PCH)])
        return 0

    lax.fori_loop(0, NS // PCH, out, 0)


def _make_hops(width):
    halves = width // 16

    @functools.partial(
        pl.kernel,
        out_type=(
            jax.ShapeDtypeStruct((BB * NP, width), jnp.float32),
            jax.ShapeDtypeStruct((BB * NP, width), jnp.float32),
        ),
        mesh=_mesh,
        scratch_types=[
            pltpu.VMEM_SHARED((NP, width), jnp.float32),
            pltpu.VMEM((SCH, 128), jnp.int32),
            pltpu.VMEM((SCH, 128), jnp.int32),
            pltpu.VMEM((SCH * 128, width), jnp.float32),
            pltpu.VMEM((PCH, width), jnp.float32),
            pltpu.VMEM((PCH, width), jnp.float32),
            pltpu.VMEM((PCH, width), jnp.float32),
            pltpu.VMEM((PCH, width), jnp.float32),
            pltpu.SemaphoreType.DMA,
            pltpu.SemaphoreType.DMA,
        ],
        compiler_params=pltpu.CompilerParams(use_tc_tiling_on_sc=False),
    )
    def hops(p_init, q2, q1, d2x, d1x, rowB, col2d, z, ptmp,
             acc, rbuf, cbuf, gbuf, abuf, dbuf, qbuf, zbuf,
             gsem, ssem):
        c = lax.axis_index("c")
        s = lax.axis_index("s")
        nbase = s * NS
        _fill(zbuf, PCH, width, 0.0)

        def zero_acc(i, _):
            pltpu.sync_copy(zbuf, acc.at[pl.ds(nbase + i * PCH, PCH)])
            return 0

        def edge_phase(tab):
            ebase = c * (EP // 128) + s * CPT

            def stage(t, _):
                pltpu.sync_copy(rowB.at[pl.ds(ebase + t * SCH, SCH)], rbuf)
                pltpu.sync_copy(col2d.at[pl.ds(s * CPT + t * SCH, SCH)], cbuf)
                gs = [
                    pltpu.async_copy(
                        tab.at[rbuf.at[j]], gbuf.at[pl.ds(j * 128, 128)], gsem)
                    for j in range(SCH)
                ]
                for h in gs:
                    h.wait()
                ss = [
                    pltpu.async_copy(
                        gbuf.at[pl.ds(j * 128, 128)], acc.at[cbuf.at[j]],
                        ssem, add=True)
                    for j in range(SCH)
                ]
                for h in ss:
                    h.wait()
                return 0

            lax.fori_loop(0, CPT // SCH, stage, 0)

        def post(qref, dref, dst):
            def chunk(i, _):
                g = nbase + i * PCH
                pltpu.sync_copy(acc.at[pl.ds(g, PCH)], abuf)
                pltpu.sync_copy(zbuf, acc.at[pl.ds(g, PCH)])
                pltpu.sync_copy(dref.at[pl.ds(g, PCH)], dbuf)
                if qref is not None:
                    pltpu.sync_copy(qref.at[pl.ds(c * NP + g, PCH)], qbuf)

                def rowfn(r, _):
                    for h in range(halves):
                        sl = pl.ds(h * 16, 16)
                        v = abuf[r, sl] * dbuf[r, sl]
                        if qref is not None:
                            v = v + qbuf[r, sl]
                        abuf[r, sl] = v
                    return 0

                lax.fori_loop(0, PCH, rowfn, 0)
                pltpu.sync_copy(abuf, dst.at[pl.ds(c * NP + g, PCH)])
                return 0

            lax.fori_loop(0, NS // PCH, chunk, 0)

        lax.fori_loop(0, NS // PCH, zero_acc, 0)
        plsc.subcore_barrier()
        edge_phase(p_init)
        plsc.subcore_barrier()
        post(q2, d2x, ptmp)
        plsc.subcore_barrier()
        edge_phase(ptmp)
        plsc.subcore_barrier()
        post(q1, d2x, ptmp)
        plsc.subcore_barrier()
        edge_phase(ptmp)
        plsc.subcore_barrier()
        post(None, d1x, z)

    return hops


_hops32 = _make_hops(32)
_hops16 = _make_hops(16)


def _prep1_body(x_ref, degp_ref, wc_ref,
                y0, q1, q2, p3, d2x, d1x, d2x16, d1x16, dis_o):
    deg = degp_ref[0, :, 0] + degp_ref[1, :, 0]
    dis = jnp.where(deg > 0, lax.rsqrt(jnp.maximum(deg, 1e-12)), 0.0)
    dis = dis.reshape(BLK, 1)
    xw = jnp.dot(x_ref[...], wc_ref[...], preferred_element_type=jnp.float32)
    y0[...] = xw[:, 0:32]
    q1[...] = dis * xw[:, 32:64]
    q2[...] = dis * xw[:, 64:96]
    p3[...] = dis * xw[:, 96:128]
    d2 = dis * dis
    d2x[...] = jnp.broadcast_to(d2, (BLK, 32))
    d1x[...] = jnp.broadcast_to(dis, (BLK, 32))
    d2x16[...] = jnp.broadcast_to(d2, (BLK, 16))
    d1x16[...] = jnp.broadcast_to(dis, (BLK, 16))
    dis_o[...] = dis


def _prep1(xf, degp2, wc1):
    nb = NP // BLK
    fo32 = jax.ShapeDtypeStruct((BB * NP, 32), jnp.float32)
    n32 = jax.ShapeDtypeStruct((NP, 32), jnp.float32)
    n16 = jax.ShapeDtypeStruct((NP, 16), jnp.float32)
    bs_bn = pl.BlockSpec((BLK, 32), lambda i: (i, 0))
    bs_n32 = pl.BlockSpec((BLK, 32), lambda i: (i % nb, 0))
    bs_n16 = pl.BlockSpec((BLK, 16), lambda i: (i % nb, 0))
    return pl.pallas_call(
        _prep1_body,
        grid=(BB * NP // BLK,),
        in_specs=[
            bs_bn,
            pl.BlockSpec((2, BLK, 16), lambda i: (0, i % nb, 0)),
            pl.BlockSpec((32, 128), lambda i: (0, 0)),
        ],
        out_specs=[bs_bn, bs_bn, bs_bn, bs_bn,
                   bs_n32, bs_n32, bs_n16, bs_n16,
                   pl.BlockSpec((BLK, 1), lambda i: (i % nb, 0))],
        out_shape=[fo32, fo32, fo32, fo32, n32, n32, n16, n16,
                   jax.ShapeDtypeStruct((NP, 1), jnp.float32)],
    )(xf, degp2, wc1)


def _mid_body(wo, y0_ref, z_ref, g_ref, be_ref, dis_ref, b_ref, wc_ref,
              y0n, q1n, q2n, p3n):
    fo = wo // 4
    t = y0_ref[...] + z_ref[...] + b_ref[...]
    m = jnp.mean(t, axis=0, keepdims=True)
    v = jnp.mean((t - m) ** 2, axis=0, keepdims=True)
    h = g_ref[...] * (t - m) / jnp.sqrt(v + 1e-5) + be_ref[...]
    h = jnp.where(h >= 0, h, 0.01 * h)
    hw = jnp.dot(h.reshape(BB * BLK, 32), wc_ref[...],
                 preferred_element_type=jnp.float32).reshape(BB, BLK, wo)
    dis = dis_ref[...].reshape(1, BLK, 1)
    y0n[...] = hw[..., 0:fo]
    q1n[...] = dis * hw[..., fo:2 * fo]
    q2n[...] = dis * hw[..., 2 * fo:3 * fo]
    p3n[...] = dis * hw[..., 3 * fo:4 * fo]


def _mid(y0, zz, gp, bep, dis, br, wc):
    wo = wc.shape[1]
    fo = wo // 4
    out = jax.ShapeDtypeStruct((BB, NP, fo), jnp.float32)
    bs_o = pl.BlockSpec((BB, BLK, fo), lambda i: (0, i, 0))
    return pl.pallas_call(
        functools.partial(_mid_body, wo),
        grid=(NP // BLK,),
        in_specs=[
            pl.BlockSpec((BB, BLK, 32), lambda i: (0, i, 0)),
            pl.BlockSpec((BB, BLK, 32), lambda i: (0, i, 0)),
            pl.BlockSpec((BLK, 32), lambda i: (i, 0)),
            pl.BlockSpec((BLK, 32), lambda i: (i, 0)),
            pl.BlockSpec((BLK, 1), lambda i: (i, 0)),
            pl.BlockSpec((1, 32), lambda i: (0, 0)),
            pl.BlockSpec((32, wo), lambda i: (0, 0)),
        ],
        out_specs=[bs_o, bs_o, bs_o, bs_o],
        out_shape=[out, out, out, out],
    )(y0, zz, gp, bep, dis, br, wc)


def _final_body(y0_ref, z_ref, b_ref, vmin_ref, vmax_ref, o_ref):
    t = y0_ref[...] + z_ref[...] + b_ref[...]
    sg = jax.nn.sigmoid(t * 0.1)
    a = vmin_ref[...][None]
    b = vmax_ref[...][None]
    o_ref[...] = a + (b - a) * sg


def _final(y0, zz, b3r, vminp, vmaxp):
    return pl.pallas_call(
        _final_body,
        grid=(NP // BLK,),
        in_specs=[
            pl.BlockSpec((BB, BLK, 16), lambda i: (0, i, 0)),
            pl.BlockSpec((BB, BLK, 16), lambda i: (0, i, 0)),
            pl.BlockSpec((1, 16), lambda i: (0, 0)),
            pl.BlockSpec((BLK, 16), lambda i: (i, 0)),
            pl.BlockSpec((BLK, 16), lambda i: (i, 0)),
        ],
        out_specs=pl.BlockSpec((BB, BLK, 16), lambda i: (0, i, 0)),
        out_shape=jax.ShapeDtypeStruct((BB, NP, 16), jnp.float32),
    )(y0, zz, b3r, vminp, vmaxp)


@jax.jit
def kernel(x, edge_index, W1, b1, W2, b2, W3, b3, g1, be1, g2, be2,
           val_min, val_max):
    f32 = jnp.float32
    row = edge_index[0]
    col = edge_index[1]
    rowp = jnp.concatenate([row, jnp.full((EP - EE,), NN, jnp.int32)])
    colp = jnp.concatenate([col, jnp.full((EP - EE,), NN + 1, jnp.int32)])
    rowB = jnp.concatenate([rowp, rowp + NP]).reshape(2 * (EP // 128), 128)
    col2d = colp.reshape(EP // 128, 128)

    xf = jnp.pad(x, ((0, 0), (0, NP - NN), (0, 0))).reshape(BB * NP, 32)
    wc1 = jnp.concatenate([W1[0], W1[1], W1[2], W1[3]], axis=1)
    wc2 = jnp.concatenate([W2[0], W2[1], W2[2], W2[3]], axis=1)
    w3p = jnp.pad(W3, ((0, 0), (0, 0), (0, 13)))
    wc3 = jnp.concatenate([w3p[0], w3p[1], w3p[2], w3p[3]], axis=1)
    g1p = jnp.pad(g1.reshape(NN, 32), ((0, NP - NN), (0, 0)))
    be1p = jnp.pad(be1.reshape(NN, 32), ((0, NP - NN), (0, 0)))
    g2p = jnp.pad(g2.reshape(NN, 32), ((0, NP - NN), (0, 0)))
    be2p = jnp.pad(be2.reshape(NN, 32), ((0, NP - NN), (0, 0)))
    vminp = jnp.pad(val_min, ((0, NP - NN), (0, 13)))
    vmaxp = jnp.pad(val_max, ((0, NP - NN), (0, 13)))
    b1r = b1.reshape(1, 32)
    b2r = b2.reshape(1, 32)
    b3r = jnp.pad(b3, (0, 13)).reshape(1, 16)

    degp = _deg_kernel(col2d)
    y0a, q1a, q2a, p3a, d2x, d1x, d2x16, d1x16, dis = _prep1(
        xf, degp.reshape(2, NP, 16), wc1)

    z1, _ = _hops32(p3a, q2a, q1a, d2x, d1x, rowB, col2d)

    y0b, q1b, q2b, p3b = _mid(y0a.reshape(BB, NP, 32), z1.reshape(BB, NP, 32),
                              g1p, be1p, dis, b1r, wc2)
    z2, _ = _hops32(p3b.reshape(BB * NP, 32), q2b.reshape(BB * NP, 32),
                    q1b.reshape(BB * NP, 32), d2x, d1x, rowB, col2d)

    y0c, q1c, q2c, p3c = _mid(y0b, z2.reshape(BB, NP, 32),
                              g2p, be2p, dis, b2r, wc3)
    z3, _ = _hops16(p3c.reshape(BB * NP, 16), q2c.reshape(BB * NP, 16),
                    q1c.reshape(BB * NP, 16), d2x16, d1x16, rowB, col2d)

    res = _final(y0c, z3.reshape(BB, NP, 16), b3r, vminp, vmaxp)
    return res[:, :NN, :3]


# speedup vs baseline: 165.1168x; 1.0005x over previous
import functools

import jax
import jax.numpy as jnp
from jax import lax
from jax.experimental import pallas as pl
from jax.experimental.pallas import tpu as pltpu
from jax.experimental.pallas import tpu_sc as plsc

NN = 50000
EE = 800000
BB = 2
NP = 51200
EP = 802816
NS = NP // 16
CPT = EP // 128 // 16
PCH = 64
SCH = 4
BLK = 1024

_mesh = plsc.VectorSubcoreMesh(core_axis_name="c", subcore_axis_name="s")


def _fill(ref, rows, width, val):
    v = jnp.full((16,), val, jnp.float32)

    def body(r, _):
        for h in range(width // 16):
            ref[r, pl.ds(h * 16, 16)] = v
        return 0

    lax.fori_loop(0, rows, body, 0)


@functools.partial(
    pl.kernel,
    out_type=jax.ShapeDtypeStruct((2 * NP, 16), jnp.float32),
    mesh=_mesh,
    scratch_types=[
        pltpu.VMEM_SHARED((NP, 16), jnp.float32),
        pltpu.VMEM((512,), jnp.int32),
        pltpu.VMEM((512, 16), jnp.float32),
        pltpu.VMEM((PCH, 16), jnp.float32),
        pltpu.SemaphoreType.DMA,
    ],
    compiler_params=pltpu.CompilerParams(use_tc_tiling_on_sc=False),
)
def _deg_kernel(col2d, degp, dacc, cbuf, ones, zb, sem):
    c = lax.axis_index("c")
    s = lax.axis_index("s")
    _fill(ones, 512, 16, 1.0)
    _fill(zb, PCH, 16, 0.0)

    def zero(i, _):
        pltpu.sync_copy(zb, dacc.at[pl.ds(s * NS + i * PCH, PCH)])
        return 0

    lax.fori_loop(0, NS // PCH, zero, 0)
    plsc.subcore_barrier()

    base = c * (EP // 2) + s * (EP // 32)

    def stage(t, _):
        pltpu.sync_copy(col2d.at[pl.ds(base + t * 512, 512)], cbuf)
        pltpu.async_copy(ones, dacc.at[cbuf], sem, add=True).wait()
        return 0

    lax.fori_loop(0, 49, stage, 0)
    plsc.subcore_barrier()

    def out(i, _):
        g = s * NS + i * PCH
        pltpu.sync_copy(dacc.at[pl.ds(g, PCH)], zb)
        pltpu.sync_copy(zb, degp.at[pl.ds(c * NP + g, PCH)])
        return 0

    lax.fori_loop(0, NS // PCH, out, 0)


def _make_hops(width):
    halves = width // 16

    @functools.partial(
        pl.kernel,
        out_type=(
            jax.ShapeDtypeStruct((BB * NP, width), jnp.float32),
            jax.ShapeDtypeStruct((BB * NP, width), jnp.float32),
        ),
        mesh=_mesh,
        scratch_types=[
            pltpu.VMEM_SHARED((NP, width), jnp.float32),
            pltpu.VMEM((SCH * 128,), jnp.int32),
            pltpu.VMEM((SCH * 128,), jnp.int32),
            pltpu.VMEM((SCH * 128, width), jnp.float32),
            pltpu.VMEM((PCH, width), jnp.float32),
            pltpu.VMEM((PCH, width), jnp.float32),
            pltpu.VMEM((PCH, width), jnp.float32),
            pltpu.VMEM((PCH, width), jnp.float32),
            pltpu.SemaphoreType.DMA,
            pltpu.SemaphoreType.DMA,
        ],
        compiler_params=pltpu.CompilerParams(use_tc_tiling_on_sc=False),
    )
    def hops(p_init, q2, q1, d2x, d1x, rowB, col2d, z, ptmp,
             acc, rbuf, cbuf, gbuf, abuf, dbuf, qbuf, zbuf,
             gsem, ssem):
        c = lax.axis_index("c")
        s = lax.axis_index("s")
        nbase = s * NS
        _fill(zbuf, PCH, width, 0.0)

        def zero_acc(i, _):
            pltpu.sync_copy(zbuf, acc.at[pl.ds(nbase + i * PCH, PCH)])
            return 0

        def edge_phase(tab):
            ebase = c * EP + s * (CPT * 128)

            def stage(t, _):
                pltpu.sync_copy(rowB.at[pl.ds(ebase + t * (SCH * 128), SCH * 128)], rbuf)
                pltpu.sync_copy(col2d.at[pl.ds(s * (CPT * 128) + t * (SCH * 128), SCH * 128)], cbuf)
                pltpu.async_copy(tab.at[rbuf], gbuf, gsem).wait()
                pltpu.async_copy(gbuf, acc.at[cbuf], ssem, add=True).wait()
                return 0

            lax.fori_loop(0, CPT // SCH, stage, 0)

        def post(qref, dref, dst):
            def chunk(i, _):
                g = nbase + i * PCH
                pltpu.sync_copy(acc.at[pl.ds(g, PCH)], abuf)
                pltpu.sync_copy(zbuf, acc.at[pl.ds(g, PCH)])
                pltpu.sync_copy(dref.at[pl.ds(g, PCH)], dbuf)
                if qref is not None:
                    pltpu.sync_copy(qref.at[pl.ds(c * NP + g, PCH)], qbuf)

                def rowfn(r, _):
                    for h in range(halves):
                        sl = pl.ds(h * 16, 16)
                        v = abuf[r, sl] * dbuf[r, sl]
                        if qref is not None:
                            v = v + qbuf[r, sl]
                        abuf[r, sl] = v
                    return 0

                lax.fori_loop(0, PCH, rowfn, 0)
                pltpu.sync_copy(abuf, dst.at[pl.ds(c * NP + g, PCH)])
                return 0

            lax.fori_loop(0, NS // PCH, chunk, 0)

        lax.fori_loop(0, NS // PCH, zero_acc, 0)
        plsc.subcore_barrier()
        edge_phase(p_init)
        plsc.subcore_barrier()
        post(q2, d2x, ptmp)
        plsc.subcore_barrier()
        edge_phase(ptmp)
        plsc.subcore_barrier()
        post(q1, d2x, ptmp)
        plsc.subcore_barrier()
        edge_phase(ptmp)
        plsc.subcore_barrier()
        post(None, d1x, z)

    return hops


_hops32 = _make_hops(32)
_hops16 = _make_hops(16)


def _prep1_body(x_ref, degp_ref, wc_ref,
                y0, q1, q2, p3, d2x, d1x, d2x16, d1x16, dis_o):
    deg = degp_ref[0, :, 0] + degp_ref[1, :, 0]
    dis = jnp.where(deg > 0, lax.rsqrt(jnp.maximum(deg, 1e-12)), 0.0)
    dis = dis.reshape(BLK, 1)
    xw = jnp.dot(x_ref[...], wc_ref[...], preferred_element_type=jnp.float32)
    y0[...] = xw[:, 0:32]
    q1[...] = dis * xw[:, 32:64]
    q2[...] = dis * xw[:, 64:96]
    p3[...] = dis * xw[:, 96:128]
    d2 = dis * dis
    d2x[...] = jnp.broadcast_to(d2, (BLK, 32))
    d1x[...] = jnp.broadcast_to(dis, (BLK, 32))
    d2x16[...] = jnp.broadcast_to(d2, (BLK, 16))
    d1x16[...] = jnp.broadcast_to(dis, (BLK, 16))
    dis_o[...] = dis


def _prep1(xf, degp2, wc1):
    nb = NP // BLK
    fo32 = jax.ShapeDtypeStruct((BB * NP, 32), jnp.float32)
    n32 = jax.ShapeDtypeStruct((NP, 32), jnp.float32)
    n16 = jax.ShapeDtypeStruct((NP, 16), jnp.float32)
    bs_bn = pl.BlockSpec((BLK, 32), lambda i: (i, 0))
    bs_n32 = pl.BlockSpec((BLK, 32), lambda i: (i % nb, 0))
    bs_n16 = pl.BlockSpec((BLK, 16), lambda i: (i % nb, 0))
    return pl.pallas_call(
        _prep1_body,
        grid=(BB * NP // BLK,),
        in_specs=[
            bs_bn,
            pl.BlockSpec((2, BLK, 16), lambda i: (0, i % nb, 0)),
            pl.BlockSpec((32, 128), lambda i: (0, 0)),
        ],
        out_specs=[bs_bn, bs_bn, bs_bn, bs_bn,
                   bs_n32, bs_n32, bs_n16, bs_n16,
                   pl.BlockSpec((BLK, 1), lambda i: (i % nb, 0))],
        out_shape=[fo32, fo32, fo32, fo32, n32, n32, n16, n16,
                   jax.ShapeDtypeStruct((NP, 1), jnp.float32)],
    )(xf, degp2, wc1)


def _mid_body(wo, y0_ref, z_ref, g_ref, be_ref, dis_ref, b_ref, wc_ref,
              y0n, q1n, q2n, p3n):
    fo = wo // 4
    t = y0_ref[...] + z_ref[...] + b_ref[...]
    m = jnp.mean(t, axis=0, keepdims=True)
    v = jnp.mean((t - m) ** 2, axis=0, keepdims=True)
    h = g_ref[...] * (t - m) / jnp.sqrt(v + 1e-5) + be_ref[...]
    h = jnp.where(h >= 0, h, 0.01 * h)
    hw = jnp.dot(h.reshape(BB * BLK, 32), wc_ref[...],
                 preferred_element_type=jnp.float32).reshape(BB, BLK, wo)
    dis = dis_ref[...].reshape(1, BLK, 1)
    y0n[...] = hw[..., 0:fo]
    q1n[...] = dis * hw[..., fo:2 * fo]
    q2n[...] = dis * hw[..., 2 * fo:3 * fo]
    p3n[...] = dis * hw[..., 3 * fo:4 * fo]


def _mid(y0, zz, gp, bep, dis, br, wc):
    wo = wc.shape[1]
    fo = wo // 4
    out = jax.ShapeDtypeStruct((BB, NP, fo), jnp.float32)
    bs_o = pl.BlockSpec((BB, BLK, fo), lambda i: (0, i, 0))
    return pl.pallas_call(
        functools.partial(_mid_body, wo),
        grid=(NP // BLK,),
        in_specs=[
            pl.BlockSpec((BB, BLK, 32), lambda i: (0, i, 0)),
            pl.BlockSpec((BB, BLK, 32), lambda i: (0, i, 0)),
            pl.BlockSpec((BLK, 32), lambda i: (i, 0)),
            pl.BlockSpec((BLK, 32), lambda i: (i, 0)),
            pl.BlockSpec((BLK, 1), lambda i: (i, 0)),
            pl.BlockSpec((1, 32), lambda i: (0, 0)),
            pl.BlockSpec((32, wo), lambda i: (0, 0)),
        ],
        out_specs=[bs_o, bs_o, bs_o, bs_o],
        out_shape=[out, out, out, out],
    )(y0, zz, gp, bep, dis, br, wc)


def _final_body(y0_ref, z_ref, b_ref, vmin_ref, vmax_ref, o_ref):
    t = y0_ref[...] + z_ref[...] + b_ref[...]
    sg = jax.nn.sigmoid(t * 0.1)
    a = vmin_ref[...][None]
    b = vmax_ref[...][None]
    o_ref[...] = a + (b - a) * sg


def _final(y0, zz, b3r, vminp, vmaxp):
    return pl.pallas_call(
        _final_body,
        grid=(NP // BLK,),
        in_specs=[
            pl.BlockSpec((BB, BLK, 16), lambda i: (0, i, 0)),
            pl.BlockSpec((BB, BLK, 16), lambda i: (0, i, 0)),
            pl.BlockSpec((1, 16), lambda i: (0, 0)),
            pl.BlockSpec((BLK, 16), lambda i: (i, 0)),
            pl.BlockSpec((BLK, 16), lambda i: (i, 0)),
        ],
        out_specs=pl.BlockSpec((BB, BLK, 16), lambda i: (0, i, 0)),
        out_shape=jax.ShapeDtypeStruct((BB, NP, 16), jnp.float32),
    )(y0, zz, b3r, vminp, vmaxp)


@jax.jit
def kernel(x, edge_index, W1, b1, W2, b2, W3, b3, g1, be1, g2, be2,
           val_min, val_max):
    f32 = jnp.float32
    row = edge_index[0]
    col = edge_index[1]
    rowp = jnp.concatenate([row, jnp.full((EP - EE,), NN, jnp.int32)])
    colp = jnp.concatenate([col, jnp.full((EP - EE,), NN + 1, jnp.int32)])
    rowB = jnp.concatenate([rowp, rowp + NP])
    col2d = colp

    xf = jnp.pad(x, ((0, 0), (0, NP - NN), (0, 0))).reshape(BB * NP, 32)
    wc1 = jnp.concatenate([W1[0], W1[1], W1[2], W1[3]], axis=1)
    wc2 = jnp.concatenate([W2[0], W2[1], W2[2], W2[3]], axis=1)
    w3p = jnp.pad(W3, ((0, 0), (0, 0), (0, 13)))
    wc3 = jnp.concatenate([w3p[0], w3p[1], w3p[2], w3p[3]], axis=1)
    g1p = jnp.pad(g1.reshape(NN, 32), ((0, NP - NN), (0, 0)))
    be1p = jnp.pad(be1.reshape(NN, 32), ((0, NP - NN), (0, 0)))
    g2p = jnp.pad(g2.reshape(NN, 32), ((0, NP - NN), (0, 0)))
    be2p = jnp.pad(be2.reshape(NN, 32), ((0, NP - NN), (0, 0)))
    vminp = jnp.pad(val_min, ((0, NP - NN), (0, 13)))
    vmaxp = jnp.pad(val_max, ((0, NP - NN), (0, 13)))
    b1r = b1.reshape(1, 32)
    b2r = b2.reshape(1, 32)
    b3r = jnp.pad(b3, (0, 13)).reshape(1, 16)

    degp = _deg_kernel(col2d)
    y0a, q1a, q2a, p3a, d2x, d1x, d2x16, d1x16, dis = _prep1(
        xf, degp.reshape(2, NP, 16), wc1)

    z1, _ = _hops32(p3a, q2a, q1a, d2x, d1x, rowB, col2d)

    y0b, q1b, q2b, p3b = _mid(y0a.reshape(BB, NP, 32), z1.reshape(BB, NP, 32),
                              g1p, be1p, dis, b1r, wc2)
    z2, _ = _hops32(p3b.reshape(BB * NP, 32), q2b.reshape(BB * NP, 32),
                    q1b.reshape(BB * NP, 32), d2x, d1x, rowB, col2d)

    y0c, q1c, q2c, p3c = _mid(y0b, z2.reshape(BB, NP, 32),
                              g2p, be2p, dis, b2r, wc3)
    z3, _ = _hops16(p3c.reshape(BB * NP, 16), q2c.reshape(BB * NP, 16),
                    q1c.reshape(BB * NP, 16), d2x16, d1x16, rowB, col2d)

    res = _final(y0c, z3.reshape(BB, NP, 16), b3r, vminp, vmaxp)
    return res[:, :NN, :3]
